# baseline (speedup 1.0000x reference)
import functools

import jax
import jax.numpy as jnp
from jax import lax
from jax.experimental import pallas as pl
from jax.experimental.pallas import tpu as pltpu
from jax.experimental.pallas import tpu_sc as plsc

NC = 2
NS = 16
L = 16


def _gather_body(b_per_w, idx_hbm, mem_hbm, h_hbm, idx_v, rows_v, sem):
    sid = lax.axis_index("s")
    base = sid * b_per_w
    pltpu.sync_copy(idx_hbm.at[pl.ds(base, b_per_w)], idx_v)
    pltpu.async_copy(mem_hbm.at[idx_v], rows_v, sem).wait()
    pltpu.sync_copy(rows_v, h_hbm.at[pl.ds(base, b_per_w)])


def _sc_gather(idx, mem):
    B = idx.shape[0]
    D = mem.shape[1]
    b_per_w = B // NS
    mesh = plsc.VectorSubcoreMesh(
        core_axis_name="c", subcore_axis_name="s", num_cores=1)
    kern = pl.kernel(
        functools.partial(_gather_body, b_per_w),
        out_type=jax.ShapeDtypeStruct((B, D), jnp.float32),
        mesh=mesh,
        compiler_params=pltpu.CompilerParams(
            use_tc_tiling_on_sc=False, needs_layout_passes=False),
        scratch_types=[
            pltpu.VMEM((b_per_w,), jnp.int32),
            pltpu.VMEM((b_per_w, D), jnp.float32),
            pltpu.SemaphoreType.DMA,
        ],
    )
    return kern(idx, mem)


def _gru_body(h_ref, val_ref, wi_ref, wh_ref, bi_ref, bh_ref, o_ref):
    d = h_ref.shape[1]
    h = h_ref[...]
    gi = jnp.dot(val_ref[...], wi_ref[...],
                 preferred_element_type=jnp.float32) + bi_ref[...]
    gh = jnp.dot(h, wh_ref[...],
                 preferred_element_type=jnp.float32) + bh_ref[...]
    r = jax.nn.sigmoid(gi[:, :d] + gh[:, :d])
    z = jax.nn.sigmoid(gi[:, d:2 * d] + gh[:, d:2 * d])
    n = jnp.tanh(gi[:, 2 * d:] + r * gh[:, 2 * d:])
    o_ref[...] = (1.0 - z) * n + z * h


def _tc_gru(h, val, W_i, W_h, b_i, b_h):
    B, D = h.shape
    BLK = 1024
    grid = (B // BLK,)
    return pl.pallas_call(
        _gru_body,
        grid=grid,
        in_specs=[
            pl.BlockSpec((BLK, D), lambda i: (i, 0)),
            pl.BlockSpec((BLK, D), lambda i: (i, 0)),
            pl.BlockSpec((D, 3 * D), lambda i: (0, 0)),
            pl.BlockSpec((D, 3 * D), lambda i: (0, 0)),
            pl.BlockSpec((1, 3 * D), lambda i: (0, 0)),
            pl.BlockSpec((1, 3 * D), lambda i: (0, 0)),
        ],
        out_specs=pl.BlockSpec((BLK, D), lambda i: (i, 0)),
        out_shape=jax.ShapeDtypeStruct((B, D), jnp.float32),
    )(h, val, W_i, W_h, b_i.reshape(1, 3 * D), b_h.reshape(1, 3 * D))


CH = 128


def _scatter_body(M, B, slab, mem_hbm, idx_hbm, hnew_hbm, out_hbm,
                  idxall_v, selpos_v, ctgt_v, crows_v, sem):
    cid = lax.axis_index("c")
    sid = lax.axis_index("s")
    wid = sid * NC + cid
    lo = wid * slab

    pltpu.sync_copy(mem_hbm.at[pl.ds(lo, slab)], out_hbm.at[pl.ds(lo, slab)])

    pltpu.sync_copy(idx_hbm, idxall_v)

    def filt(k, off):
        v = idxall_v[pl.ds(k * L, L)]
        m = (v >= lo) & (v < lo + slab)
        pos = k * L + lax.iota(jnp.int32, L)
        pc = plsc.cumsum(m.astype(jnp.int32))
        plsc.store_scatter(selpos_v, [off + pc - 1], pos, mask=m)
        return off + jnp.max(pc)

    count = lax.fori_loop(0, B // L, filt, jnp.int32(0), unroll=4)

    @pl.when(count > 0)
    def _pad():
        pv = plsc.load_gather(
            selpos_v, [jnp.full((L,), count - 1, jnp.int32)])
        for t in range(CH // L):
            selpos_v[pl.ds(count + t * L, L)] = pv

    def chunk(j, _):
        cslice = selpos_v.at[pl.ds(j * CH, CH)]
        pltpu.sync_copy(idx_hbm.at[cslice], ctgt_v)
        pltpu.async_copy(hnew_hbm.at[cslice], crows_v, sem).wait()
        pltpu.sync_copy(crows_v, out_hbm.at[ctgt_v])
        return 0

    nchunks = (count + CH - 1) // CH
    lax.fori_loop(0, nchunks, chunk, 0)


def _sc_copy_scatter(mem, idx, h_new):
    M, D = mem.shape
    B = idx.shape[0]
    slab = M // (NC * NS)
    mesh = plsc.VectorSubcoreMesh(core_axis_name="c", subcore_axis_name="s")
    kern = pl.kernel(
        functools.partial(_scatter_body, M, B, slab),
        out_type=jax.ShapeDtypeStruct((M, D), jnp.float32),
        mesh=mesh,
        compiler_params=pltpu.CompilerParams(
            use_tc_tiling_on_sc=False, needs_layout_passes=False),
        scratch_types=[
            pltpu.VMEM((B,), jnp.int32),
            pltpu.VMEM((B + CH + L,), jnp.int32),
            pltpu.VMEM((CH,), jnp.int32),
            pltpu.VMEM((CH, D), jnp.float32),
            pltpu.SemaphoreType.DMA,
        ],
    )
    return kern(mem, idx, h_new)


def kernel(mem, idx, val, W_i, W_h, b_i, b_h):
    h = _sc_gather(idx, mem)
    h_new = _tc_gru(h, val, W_i, W_h, b_i, b_h)
    return _sc_copy_scatter(mem, idx, h_new)


# speedup vs baseline: 5.9640x; 5.9640x over previous
import functools

import jax
import jax.numpy as jnp
from jax import lax
from jax.experimental import pallas as pl
from jax.experimental.pallas import tpu as pltpu
from jax.experimental.pallas import tpu_sc as plsc

NC = 2
NS = 16
L = 16
NW = NC * NS
DP = 128
CH = 128


def _t_fwd_body(memT_ref, o_ref):
    t = jnp.transpose(memT_ref[...])
    o_ref[...] = jnp.concatenate([t, jnp.zeros_like(t)], axis=1)


def _tc_transpose_pad(memT):
    D, M = memT.shape
    BW = 1024
    grid = (pl.cdiv(M, BW),)
    return pl.pallas_call(
        _t_fwd_body,
        grid=grid,
        in_specs=[pl.BlockSpec((D, BW), lambda i: (0, i))],
        out_specs=pl.BlockSpec((BW, 2 * D), lambda i: (i, 0)),
        out_shape=jax.ShapeDtypeStruct((M, 2 * D), jnp.float32),
    )(memT)


def _gather_body(b_per_w, idx_hbm, pad_hbm, h_hbm, idx_v, rows_v, sem):
    cid = lax.axis_index("c")
    sid = lax.axis_index("s")
    wid = sid * NC + cid
    base = wid * b_per_w
    pltpu.sync_copy(idx_hbm.at[pl.ds(base, b_per_w)], idx_v)
    pltpu.async_copy(pad_hbm.at[idx_v], rows_v, sem).wait()
    pltpu.sync_copy(rows_v, h_hbm.at[pl.ds(base, b_per_w)])


def _sc_gather(idx, mem_pad):
    B = idx.shape[0]
    b_per_w = B // NW
    mesh = plsc.VectorSubcoreMesh(core_axis_name="c", subcore_axis_name="s")
    kern = pl.kernel(
        functools.partial(_gather_body, b_per_w),
        out_type=jax.ShapeDtypeStruct((B, DP), jnp.float32),
        mesh=mesh,
        compiler_params=pltpu.CompilerParams(needs_layout_passes=False),
        scratch_types=[
            pltpu.VMEM((b_per_w,), jnp.int32),
            pltpu.VMEM((b_per_w, DP), jnp.float32),
            pltpu.SemaphoreType.DMA,
        ],
    )
    return kern(idx, mem_pad)


def _gru_body(h_ref, valT_ref, wi_ref, wh_ref, bi_ref, bh_ref, o_ref):
    d = wi_ref.shape[0]
    h = h_ref[...][:, :d]
    v = jnp.transpose(valT_ref[...])
    gi = jnp.dot(v, wi_ref[...], preferred_element_type=jnp.float32) + bi_ref[...]
    gh = jnp.dot(h, wh_ref[...], preferred_element_type=jnp.float32) + bh_ref[...]
    r = jax.nn.sigmoid(gi[:, :d] + gh[:, :d])
    z = jax.nn.sigmoid(gi[:, d:2 * d] + gh[:, d:2 * d])
    n = jnp.tanh(gi[:, 2 * d:] + r * gh[:, 2 * d:])
    hn = (1.0 - z) * n + z * h
    o_ref[...] = jnp.concatenate([hn, jnp.zeros_like(hn)], axis=1)


def _tc_gru(h_pad, valT, W_i, W_h, b_i, b_h):
    B = h_pad.shape[0]
    D = W_i.shape[0]
    BLK = 1024
    grid = (B // BLK,)
    return pl.pallas_call(
        _gru_body,
        grid=grid,
        in_specs=[
            pl.BlockSpec((BLK, DP), lambda i: (i, 0)),
            pl.BlockSpec((D, BLK), lambda i: (0, i)),
            pl.BlockSpec((D, 3 * D), lambda i: (0, 0)),
            pl.BlockSpec((D, 3 * D), lambda i: (0, 0)),
            pl.BlockSpec((1, 3 * D), lambda i: (0, 0)),
            pl.BlockSpec((1, 3 * D), lambda i: (0, 0)),
        ],
        out_specs=pl.BlockSpec((BLK, DP), lambda i: (i, 0)),
        out_shape=jax.ShapeDtypeStruct((B, DP), jnp.float32),
    )(h_pad, valT, W_i, W_h, b_i.reshape(1, 3 * D), b_h.reshape(1, 3 * D))


def _scatter_body(M, B, slab, idx_hbm, hnew_hbm, buf_ref,
                  idxall_v, selpos_v, seltgt_v, winpos_v, alocal_v,
                  ctgt_v, crows_v, sem):
    cid = lax.axis_index("c")
    sid = lax.axis_index("s")
    wid = sid * NC + cid
    lo = wid * slab

    pltpu.sync_copy(idx_hbm, idxall_v.at[pl.ds(0, B)])

    def filt(k, off):
        v = idxall_v[pl.ds(k * L, L)]
        m = (v >= lo) & (v < lo + slab)
        pos = k * L + lax.iota(jnp.int32, L)
        pc = plsc.cumsum(m.astype(jnp.int32))
        plsc.store_scatter(selpos_v, [off + pc - 1], pos, mask=m)
        return off + jnp.max(pc)

    count = lax.fori_loop(0, B // L, filt, jnp.int32(0), unroll=4)

    @pl.when(count > 0)
    def _work():
        pv = plsc.load_gather(selpos_v, [jnp.full((L,), count - 1, jnp.int32)])
        for t in range(CH // L + 1):
            selpos_v[pl.ds(count + t * L, L)] = pv

        nv = (count + CH + L - 1) // L

        def canon1(i, _):
            p = selpos_v[pl.ds(i * L, L)]
            tv = plsc.load_gather(idxall_v, [p]) - lo
            seltgt_v[pl.ds(i * L, L)] = tv
            plsc.store_scatter(alocal_v, [tv], p)
            return 0

        lax.fori_loop(0, nv, canon1, 0)

        def canon2(i, _):
            tv = seltgt_v[pl.ds(i * L, L)]
            winpos_v[pl.ds(i * L, L)] = plsc.load_gather(alocal_v, [tv])
            return 0

        lax.fori_loop(0, nv, canon2, 0)

        def chunk(j, _):
            for k in range(CH // L):
                ctgt_v[pl.ds(k * L, L)] = (
                    seltgt_v[pl.ds(j * CH + k * L, L)] + lo)
            cw = winpos_v.at[pl.ds(j * CH, CH)]
            pltpu.async_copy(hnew_hbm.at[cw], crows_v, sem).wait()
            pltpu.sync_copy(crows_v, buf_ref.at[ctgt_v])
            return 0

        lax.fori_loop(0, (count + CH - 1) // CH, chunk, 0)


def _sc_scatter(buf, idx, h_new_pad):
    M = buf.shape[0]
    B = idx.shape[0]
    slab = M // NW
    mesh = plsc.VectorSubcoreMesh(core_axis_name="c", subcore_axis_name="s")
    kern = pl.kernel(
        functools.partial(_scatter_body, M, B, slab),
        out_type=(),
        mesh=mesh,
        compiler_params=pltpu.CompilerParams(needs_layout_passes=False),
        scratch_types=[
            pltpu.VMEM((B + L,), jnp.int32),
            pltpu.VMEM((B + CH + L,), jnp.int32),
            pltpu.VMEM((B + CH + L,), jnp.int32),
            pltpu.VMEM((B + CH + L,), jnp.int32),
            pltpu.VMEM((slab,), jnp.int32),
            pltpu.VMEM((CH,), jnp.int32),
            pltpu.VMEM((CH, DP), jnp.float32),
            pltpu.SemaphoreType.DMA,
        ],
    )
    kern(idx, h_new_pad, buf)


def _t_bwd_body(pad_ref, o_ref):
    d = o_ref.shape[0]
    o_ref[...] = jnp.transpose(pad_ref[...][:, :d])


def _tc_transpose_back(out_pad, D):
    M = out_pad.shape[0]
    BW = 1024
    grid = (pl.cdiv(M, BW),)
    return pl.pallas_call(
        _t_bwd_body,
        grid=grid,
        in_specs=[pl.BlockSpec((BW, 2 * D), lambda i: (i, 0))],
        out_specs=pl.BlockSpec((D, BW), lambda i: (0, i)),
        out_shape=jax.ShapeDtypeStruct((D, M), jnp.float32),
    )(out_pad)


def kernel(mem, idx, val, W_i, W_h, b_i, b_h):
    D = mem.shape[1]
    memT = jnp.transpose(mem)
    valT = jnp.transpose(val)
    mem_pad = _tc_transpose_pad(memT)
    h_pad = _sc_gather(idx, mem_pad)
    h_new_pad = _tc_gru(h_pad, valT, W_i, W_h, b_i, b_h)
    buf = jax.new_ref(mem_pad)
    _sc_scatter(buf, idx, h_new_pad)
    outT = _tc_transpose_back(buf[...], D)
    return jnp.transpose(outT)


# speedup vs baseline: 11.7544x; 1.9709x over previous
import functools

import jax
import jax.numpy as jnp
from jax import lax
from jax.experimental import pallas as pl
from jax.experimental.pallas import tpu as pltpu
from jax.experimental.pallas import tpu_sc as plsc

NC = 2
NS = 16
L = 16
NW = NC * NS
DP = 128
CH = 128


def _t_fwd_body(memT_ref, o_ref):
    t = jnp.transpose(memT_ref[...])
    o_ref[...] = jnp.concatenate([t, jnp.zeros_like(t)], axis=1)


def _tc_transpose_pad(memT):
    D, M = memT.shape
    BW = 4096
    grid = (pl.cdiv(M, BW),)
    return pl.pallas_call(
        _t_fwd_body,
        grid=grid,
        in_specs=[pl.BlockSpec((D, BW), lambda i: (0, i))],
        out_specs=pl.BlockSpec((BW, 2 * D), lambda i: (i, 0)),
        out_shape=jax.ShapeDtypeStruct((M, 2 * D), jnp.float32),
    )(memT)


def _gather_body(b_per_w, idx_hbm, pad_hbm, h_hbm, idx_v, rows_v, sem):
    cid = lax.axis_index("c")
    sid = lax.axis_index("s")
    wid = sid * NC + cid
    base = wid * b_per_w
    pltpu.sync_copy(idx_hbm.at[pl.ds(base, b_per_w)], idx_v)
    pltpu.async_copy(pad_hbm.at[idx_v], rows_v, sem).wait()
    pltpu.sync_copy(rows_v, h_hbm.at[pl.ds(base, b_per_w)])


def _sc_gather(idx, mem_pad):
    B = idx.shape[0]
    b_per_w = B // NW
    mesh = plsc.VectorSubcoreMesh(core_axis_name="c", subcore_axis_name="s")
    kern = pl.kernel(
        functools.partial(_gather_body, b_per_w),
        out_type=jax.ShapeDtypeStruct((B, DP), jnp.float32),
        mesh=mesh,
        compiler_params=pltpu.CompilerParams(needs_layout_passes=False),
        scratch_types=[
            pltpu.VMEM((b_per_w,), jnp.int32),
            pltpu.VMEM((b_per_w, DP), jnp.float32),
            pltpu.SemaphoreType.DMA,
        ],
    )
    return kern(idx, mem_pad)


def _gru_body(h_ref, valT_ref, wi_ref, wh_ref, bi_ref, bh_ref, o_ref):
    d = wi_ref.shape[0]
    h = h_ref[...][:, :d]
    v = jnp.transpose(valT_ref[...])
    gi = jnp.dot(v, wi_ref[...], preferred_element_type=jnp.float32) + bi_ref[...]
    gh = jnp.dot(h, wh_ref[...], preferred_element_type=jnp.float32) + bh_ref[...]
    r = jax.nn.sigmoid(gi[:, :d] + gh[:, :d])
    z = jax.nn.sigmoid(gi[:, d:2 * d] + gh[:, d:2 * d])
    n = jnp.tanh(gi[:, 2 * d:] + r * gh[:, 2 * d:])
    hn = (1.0 - z) * n + z * h
    o_ref[...] = jnp.concatenate([hn, jnp.zeros_like(hn)], axis=1)


def _tc_gru(h_pad, valT, W_i, W_h, b_i, b_h):
    B = h_pad.shape[0]
    D = W_i.shape[0]
    BLK = 1024
    grid = (B // BLK,)
    return pl.pallas_call(
        _gru_body,
        grid=grid,
        in_specs=[
            pl.BlockSpec((BLK, DP), lambda i: (i, 0)),
            pl.BlockSpec((D, BLK), lambda i: (0, i)),
            pl.BlockSpec((D, 3 * D), lambda i: (0, 0)),
            pl.BlockSpec((D, 3 * D), lambda i: (0, 0)),
            pl.BlockSpec((1, 3 * D), lambda i: (0, 0)),
            pl.BlockSpec((1, 3 * D), lambda i: (0, 0)),
        ],
        out_specs=pl.BlockSpec((BLK, DP), lambda i: (i, 0)),
        out_shape=jax.ShapeDtypeStruct((B, DP), jnp.float32),
    )(h_pad, valT, W_i, W_h, b_i.reshape(1, 3 * D), b_h.reshape(1, 3 * D))


def _scatter_body(M, B, slab, idx_hbm, hnew_hbm, buf_ref,
                  idxall_v, selpos_v, seltgt_v, winpos_v, alocal_v,
                  ctgt_v, crows_v, sem):
    cid = lax.axis_index("c")
    sid = lax.axis_index("s")
    wid = sid * NC + cid
    lo = wid * slab

    pltpu.sync_copy(idx_hbm, idxall_v.at[pl.ds(0, B)])

    def filt(k, off):
        v = idxall_v[pl.ds(k * L, L)]
        m = (v >= lo) & (v < lo + slab)
        pos = k * L + lax.iota(jnp.int32, L)
        pc = plsc.cumsum(m.astype(jnp.int32))
        plsc.store_scatter(selpos_v, [off + pc - 1], pos, mask=m)
        return off + jnp.max(pc)

    count = lax.fori_loop(0, B // L, filt, jnp.int32(0), unroll=4)

    @pl.when(count > 0)
    def _work():
        pv = plsc.load_gather(selpos_v, [jnp.full((L,), count - 1, jnp.int32)])
        for t in range(CH // L + 1):
            selpos_v[pl.ds(count + t * L, L)] = pv

        nv = (count + CH + L - 1) // L

        def canon1(i, _):
            p = selpos_v[pl.ds(i * L, L)]
            tv = plsc.load_gather(idxall_v, [p]) - lo
            seltgt_v[pl.ds(i * L, L)] = tv
            plsc.store_scatter(alocal_v, [tv], p)
            return 0

        lax.fori_loop(0, nv, canon1, 0)

        def canon2(i, _):
            tv = seltgt_v[pl.ds(i * L, L)]
            winpos_v[pl.ds(i * L, L)] = plsc.load_gather(alocal_v, [tv])
            return 0

        lax.fori_loop(0, nv, canon2, 0)

        def chunk(j, _):
            for k in range(CH // L):
                ctgt_v[pl.ds(k * L, L)] = (
                    seltgt_v[pl.ds(j * CH + k * L, L)] + lo)
            cw = winpos_v.at[pl.ds(j * CH, CH)]
            pltpu.async_copy(hnew_hbm.at[cw], crows_v, sem).wait()
            pltpu.sync_copy(crows_v, buf_ref.at[ctgt_v])
            return 0

        lax.fori_loop(0, (count + CH - 1) // CH, chunk, 0)


def _sc_scatter(buf, idx, h_new_pad):
    M = buf.shape[0]
    B = idx.shape[0]
    slab = M // NW
    mesh = plsc.VectorSubcoreMesh(core_axis_name="c", subcore_axis_name="s")
    kern = pl.kernel(
        functools.partial(_scatter_body, M, B, slab),
        out_type=(),
        mesh=mesh,
        compiler_params=pltpu.CompilerParams(needs_layout_passes=False),
        scratch_types=[
            pltpu.VMEM((B + L,), jnp.int32),
            pltpu.VMEM((B + CH + L,), jnp.int32),
            pltpu.VMEM((B + CH + L,), jnp.int32),
            pltpu.VMEM((B + CH + L,), jnp.int32),
            pltpu.VMEM((slab,), jnp.int32),
            pltpu.VMEM((CH,), jnp.int32),
            pltpu.VMEM((CH, DP), jnp.float32),
            pltpu.SemaphoreType.DMA,
        ],
    )
    kern(idx, h_new_pad, buf)


def _t_bwd_body(pad_ref, o_ref):
    d = o_ref.shape[0]
    o_ref[...] = jnp.transpose(pad_ref[...][:, :d])


def _tc_transpose_back(out_pad, D):
    M = out_pad.shape[0]
    BW = 4096
    grid = (pl.cdiv(M, BW),)
    return pl.pallas_call(
        _t_bwd_body,
        grid=grid,
        in_specs=[pl.BlockSpec((BW, 2 * D), lambda i: (i, 0))],
        out_specs=pl.BlockSpec((D, BW), lambda i: (0, i)),
        out_shape=jax.ShapeDtypeStruct((D, M), jnp.float32),
    )(out_pad)


def kernel(mem, idx, val, W_i, W_h, b_i, b_h):
    D = mem.shape[1]
    memT = jnp.transpose(mem)
    valT = jnp.transpose(val)
    mem_pad = _tc_transpose_pad(memT)
    h_pad = _sc_gather(idx, mem_pad)
    h_new_pad = _tc_gru(h_pad, valT, W_i, W_h, b_i, b_h)
    buf = jax.new_ref(mem_pad)
    _sc_scatter(buf, idx, h_new_pad)
    outT = _tc_transpose_back(buf[...], D)
    return jnp.transpose(outT)


# speedup vs baseline: 14.5599x; 1.2387x over previous
import functools

import jax
import jax.numpy as jnp
from jax import lax
from jax.experimental import pallas as pl
from jax.experimental.pallas import tpu as pltpu
from jax.experimental.pallas import tpu_sc as plsc

NC = 2
NS = 16
L = 16
NW = NC * NS
DP = 128
CH = 128


def _t_fwd_body(memT_ref, o_ref):
    t = jnp.transpose(memT_ref[...])
    o_ref[...] = jnp.concatenate([t, jnp.zeros_like(t)], axis=1)


def _tc_transpose_pad(memT):
    D, M = memT.shape
    BW = 8192
    grid = (pl.cdiv(M, BW),)
    return pl.pallas_call(
        _t_fwd_body,
        grid=grid,
        in_specs=[pl.BlockSpec((D, BW), lambda i: (0, i))],
        out_specs=pl.BlockSpec((BW, 2 * D), lambda i: (i, 0)),
        out_shape=jax.ShapeDtypeStruct((M, 2 * D), jnp.float32),
    )(memT)


def _gather_body(b_per_w, idx_hbm, pad_hbm, h_hbm, idx_v, rows_v, sem):
    cid = lax.axis_index("c")
    sid = lax.axis_index("s")
    wid = sid * NC + cid
    base = wid * b_per_w
    pltpu.sync_copy(idx_hbm.at[pl.ds(base, b_per_w)], idx_v)
    pltpu.async_copy(pad_hbm.at[idx_v], rows_v, sem).wait()
    pltpu.sync_copy(rows_v, h_hbm.at[pl.ds(base, b_per_w)])


def _sc_gather(idx, mem_pad):
    B = idx.shape[0]
    b_per_w = B // NW
    mesh = plsc.VectorSubcoreMesh(core_axis_name="c", subcore_axis_name="s")
    kern = pl.kernel(
        functools.partial(_gather_body, b_per_w),
        out_type=jax.ShapeDtypeStruct((B, DP), jnp.float32),
        mesh=mesh,
        compiler_params=pltpu.CompilerParams(needs_layout_passes=False),
        scratch_types=[
            pltpu.VMEM((b_per_w,), jnp.int32),
            pltpu.VMEM((b_per_w, DP), jnp.float32),
            pltpu.SemaphoreType.DMA,
        ],
    )
    return kern(idx, mem_pad)


def _gru_body(h_ref, valT_ref, wi_ref, wh_ref, bi_ref, bh_ref, o_ref):
    d = wi_ref.shape[0]
    h = h_ref[...][:, :d]
    v = jnp.transpose(valT_ref[...])
    gi = jnp.dot(v, wi_ref[...], preferred_element_type=jnp.float32) + bi_ref[...]
    gh = jnp.dot(h, wh_ref[...], preferred_element_type=jnp.float32) + bh_ref[...]
    r = jax.nn.sigmoid(gi[:, :d] + gh[:, :d])
    z = jax.nn.sigmoid(gi[:, d:2 * d] + gh[:, d:2 * d])
    n = jnp.tanh(gi[:, 2 * d:] + r * gh[:, 2 * d:])
    hn = (1.0 - z) * n + z * h
    o_ref[...] = jnp.concatenate([hn, jnp.zeros_like(hn)], axis=1)


def _tc_gru(h_pad, valT, W_i, W_h, b_i, b_h):
    B = h_pad.shape[0]
    D = W_i.shape[0]
    BLK = 1024
    grid = (B // BLK,)
    return pl.pallas_call(
        _gru_body,
        grid=grid,
        in_specs=[
            pl.BlockSpec((BLK, DP), lambda i: (i, 0)),
            pl.BlockSpec((D, BLK), lambda i: (0, i)),
            pl.BlockSpec((D, 3 * D), lambda i: (0, 0)),
            pl.BlockSpec((D, 3 * D), lambda i: (0, 0)),
            pl.BlockSpec((1, 3 * D), lambda i: (0, 0)),
            pl.BlockSpec((1, 3 * D), lambda i: (0, 0)),
        ],
        out_specs=pl.BlockSpec((BLK, DP), lambda i: (i, 0)),
        out_shape=jax.ShapeDtypeStruct((B, DP), jnp.float32),
    )(h_pad, valT, W_i, W_h, b_i.reshape(1, 3 * D), b_h.reshape(1, 3 * D))


def _scatter_body(M, B, slab, idx_hbm, hnew_hbm, buf_ref,
                  idxall_v, selpos_v, seltgt_v, winpos_v, alocal_v,
                  ctgt_v, crows_v, sem):
    cid = lax.axis_index("c")
    sid = lax.axis_index("s")
    wid = sid * NC + cid
    lo = wid * slab

    pltpu.sync_copy(idx_hbm, idxall_v.at[pl.ds(0, B)])

    def filt(k, off):
        v = idxall_v[pl.ds(k * L, L)]
        m = (v >= lo) & (v < lo + slab)
        pos = k * L + lax.iota(jnp.int32, L)
        pc = plsc.cumsum(m.astype(jnp.int32))
        plsc.store_scatter(selpos_v, [off + pc - 1], pos, mask=m)
        return off + jnp.max(pc)

    count = lax.fori_loop(0, B // L, filt, jnp.int32(0), unroll=4)

    @pl.when(count > 0)
    def _work():
        pv = plsc.load_gather(selpos_v, [jnp.full((L,), count - 1, jnp.int32)])
        for t in range(CH // L + 1):
            selpos_v[pl.ds(count + t * L, L)] = pv

        nv = (count + CH + L - 1) // L

        def canon1(i, _):
            p = selpos_v[pl.ds(i * L, L)]
            tv = plsc.load_gather(idxall_v, [p]) - lo
            seltgt_v[pl.ds(i * L, L)] = tv
            plsc.store_scatter(alocal_v, [tv], p)
            return 0

        lax.fori_loop(0, nv, canon1, 0)

        def canon2(i, _):
            tv = seltgt_v[pl.ds(i * L, L)]
            winpos_v[pl.ds(i * L, L)] = plsc.load_gather(alocal_v, [tv])
            return 0

        lax.fori_loop(0, nv, canon2, 0)

        def chunk(j, _):
            for k in range(CH // L):
                ctgt_v[pl.ds(k * L, L)] = (
                    seltgt_v[pl.ds(j * CH + k * L, L)] + lo)
            cw = winpos_v.at[pl.ds(j * CH, CH)]
            pltpu.async_copy(hnew_hbm.at[cw], crows_v, sem).wait()
            pltpu.sync_copy(crows_v, buf_ref.at[ctgt_v])
            return 0

        lax.fori_loop(0, (count + CH - 1) // CH, chunk, 0)


def _sc_scatter(buf, idx, h_new_pad):
    M = buf.shape[0]
    B = idx.shape[0]
    slab = M // NW
    mesh = plsc.VectorSubcoreMesh(core_axis_name="c", subcore_axis_name="s")
    kern = pl.kernel(
        functools.partial(_scatter_body, M, B, slab),
        out_type=(),
        mesh=mesh,
        compiler_params=pltpu.CompilerParams(needs_layout_passes=False),
        scratch_types=[
            pltpu.VMEM((B + L,), jnp.int32),
            pltpu.VMEM((B + CH + L,), jnp.int32),
            pltpu.VMEM((B + CH + L,), jnp.int32),
            pltpu.VMEM((B + CH + L,), jnp.int32),
            pltpu.VMEM((slab,), jnp.int32),
            pltpu.VMEM((CH,), jnp.int32),
            pltpu.VMEM((CH, DP), jnp.float32),
            pltpu.SemaphoreType.DMA,
        ],
    )
    kern(idx, h_new_pad, buf)


def _t_bwd_body(pad_ref, o_ref):
    d = o_ref.shape[0]
    o_ref[...] = jnp.transpose(pad_ref[...][:, :d])


def _tc_transpose_back(out_pad, D):
    M = out_pad.shape[0]
    BW = 8192
    grid = (pl.cdiv(M, BW),)
    return pl.pallas_call(
        _t_bwd_body,
        grid=grid,
        in_specs=[pl.BlockSpec((BW, 2 * D), lambda i: (i, 0))],
        out_specs=pl.BlockSpec((D, BW), lambda i: (0, i)),
        out_shape=jax.ShapeDtypeStruct((D, M), jnp.float32),
    )(out_pad)


def kernel(mem, idx, val, W_i, W_h, b_i, b_h):
    D = mem.shape[1]
    memT = jnp.transpose(mem)
    valT = jnp.transpose(val)
    mem_pad = _tc_transpose_pad(memT)
    h_pad = _sc_gather(idx, mem_pad)
    h_new_pad = _tc_gru(h_pad, valT, W_i, W_h, b_i, b_h)
    buf = jax.new_ref(mem_pad)
    _sc_scatter(buf, idx, h_new_pad)
    outT = _tc_transpose_back(buf[...], D)
    return jnp.transpose(outT)


# speedup vs baseline: 15.5310x; 1.0667x over previous
import functools

import jax
import jax.numpy as jnp
from jax import lax
from jax.experimental import pallas as pl
from jax.experimental.pallas import tpu as pltpu
from jax.experimental.pallas import tpu_sc as plsc

NC = 2
NS = 16
L = 16
NW = NC * NS
DP = 128
CH = 128


def _t_fwd_body(memT_ref, o_ref):
    t = jnp.transpose(memT_ref[...])
    o_ref[...] = jnp.concatenate([t, jnp.zeros_like(t)], axis=1)


def _tc_transpose_pad(memT):
    D, M = memT.shape
    BW = 16384
    grid = (pl.cdiv(M, BW),)
    return pl.pallas_call(
        _t_fwd_body,
        grid=grid,
        in_specs=[pl.BlockSpec((D, BW), lambda i: (0, i))],
        out_specs=pl.BlockSpec((BW, 2 * D), lambda i: (i, 0)),
        out_shape=jax.ShapeDtypeStruct((M, 2 * D), jnp.float32),
    )(memT)


def _gather_body(b_per_w, idx_hbm, pad_hbm, h_hbm, idx_v, rows_v, sem):
    cid = lax.axis_index("c")
    sid = lax.axis_index("s")
    wid = sid * NC + cid
    base = wid * b_per_w
    pltpu.sync_copy(idx_hbm.at[pl.ds(base, b_per_w)], idx_v)
    pltpu.async_copy(pad_hbm.at[idx_v], rows_v, sem).wait()
    pltpu.sync_copy(rows_v, h_hbm.at[pl.ds(base, b_per_w)])


def _sc_gather(idx, mem_pad):
    B = idx.shape[0]
    b_per_w = B // NW
    mesh = plsc.VectorSubcoreMesh(core_axis_name="c", subcore_axis_name="s")
    kern = pl.kernel(
        functools.partial(_gather_body, b_per_w),
        out_type=jax.ShapeDtypeStruct((B, DP), jnp.float32),
        mesh=mesh,
        compiler_params=pltpu.CompilerParams(needs_layout_passes=False),
        scratch_types=[
            pltpu.VMEM((b_per_w,), jnp.int32),
            pltpu.VMEM((b_per_w, DP), jnp.float32),
            pltpu.SemaphoreType.DMA,
        ],
    )
    return kern(idx, mem_pad)


def _gru_body(h_ref, valT_ref, wi_ref, wh_ref, bi_ref, bh_ref, o_ref):
    d = wi_ref.shape[0]
    h = h_ref[...][:, :d]
    v = jnp.transpose(valT_ref[...])
    gi = jnp.dot(v, wi_ref[...], preferred_element_type=jnp.float32) + bi_ref[...]
    gh = jnp.dot(h, wh_ref[...], preferred_element_type=jnp.float32) + bh_ref[...]
    r = jax.nn.sigmoid(gi[:, :d] + gh[:, :d])
    z = jax.nn.sigmoid(gi[:, d:2 * d] + gh[:, d:2 * d])
    n = jnp.tanh(gi[:, 2 * d:] + r * gh[:, 2 * d:])
    hn = (1.0 - z) * n + z * h
    o_ref[...] = jnp.concatenate([hn, jnp.zeros_like(hn)], axis=1)


def _tc_gru(h_pad, valT, W_i, W_h, b_i, b_h):
    B = h_pad.shape[0]
    D = W_i.shape[0]
    BLK = 1024
    grid = (B // BLK,)
    return pl.pallas_call(
        _gru_body,
        grid=grid,
        in_specs=[
            pl.BlockSpec((BLK, DP), lambda i: (i, 0)),
            pl.BlockSpec((D, BLK), lambda i: (0, i)),
            pl.BlockSpec((D, 3 * D), lambda i: (0, 0)),
            pl.BlockSpec((D, 3 * D), lambda i: (0, 0)),
            pl.BlockSpec((1, 3 * D), lambda i: (0, 0)),
            pl.BlockSpec((1, 3 * D), lambda i: (0, 0)),
        ],
        out_specs=pl.BlockSpec((BLK, DP), lambda i: (i, 0)),
        out_shape=jax.ShapeDtypeStruct((B, DP), jnp.float32),
    )(h_pad, valT, W_i, W_h, b_i.reshape(1, 3 * D), b_h.reshape(1, 3 * D))


def _scatter_body(M, B, slab, idx_hbm, hnew_hbm, buf_ref,
                  idxall_v, selpos_v, seltgt_v, winpos_v, alocal_v,
                  ctgt_v, crows_v, sem):
    cid = lax.axis_index("c")
    sid = lax.axis_index("s")
    wid = sid * NC + cid
    lo = wid * slab

    pltpu.sync_copy(idx_hbm, idxall_v.at[pl.ds(0, B)])

    def filt(k, off):
        v = idxall_v[pl.ds(k * L, L)]
        m = (v >= lo) & (v < lo + slab)
        pos = k * L + lax.iota(jnp.int32, L)
        pc = plsc.cumsum(m.astype(jnp.int32))
        plsc.store_scatter(selpos_v, [off + pc - 1], pos, mask=m)
        return off + jnp.max(pc)

    count = lax.fori_loop(0, B // L, filt, jnp.int32(0), unroll=4)

    @pl.when(count > 0)
    def _work():
        pv = plsc.load_gather(selpos_v, [jnp.full((L,), count - 1, jnp.int32)])
        for t in range(CH // L + 1):
            selpos_v[pl.ds(count + t * L, L)] = pv

        nv = (count + CH + L - 1) // L

        def canon1(i, _):
            p = selpos_v[pl.ds(i * L, L)]
            tv = plsc.load_gather(idxall_v, [p]) - lo
            seltgt_v[pl.ds(i * L, L)] = tv
            plsc.store_scatter(alocal_v, [tv], p)
            return 0

        lax.fori_loop(0, nv, canon1, 0)

        def canon2(i, _):
            tv = seltgt_v[pl.ds(i * L, L)]
            winpos_v[pl.ds(i * L, L)] = plsc.load_gather(alocal_v, [tv])
            return 0

        lax.fori_loop(0, nv, canon2, 0)

        def chunk(j, _):
            for k in range(CH // L):
                ctgt_v[pl.ds(k * L, L)] = (
                    seltgt_v[pl.ds(j * CH + k * L, L)] + lo)
            cw = winpos_v.at[pl.ds(j * CH, CH)]
            pltpu.async_copy(hnew_hbm.at[cw], crows_v, sem).wait()
            pltpu.sync_copy(crows_v, buf_ref.at[ctgt_v])
            return 0

        lax.fori_loop(0, (count + CH - 1) // CH, chunk, 0)


def _sc_scatter(buf, idx, h_new_pad):
    M = buf.shape[0]
    B = idx.shape[0]
    slab = M // NW
    mesh = plsc.VectorSubcoreMesh(core_axis_name="c", subcore_axis_name="s")
    kern = pl.kernel(
        functools.partial(_scatter_body, M, B, slab),
        out_type=(),
        mesh=mesh,
        compiler_params=pltpu.CompilerParams(needs_layout_passes=False),
        scratch_types=[
            pltpu.VMEM((B + L,), jnp.int32),
            pltpu.VMEM((B + CH + L,), jnp.int32),
            pltpu.VMEM((B + CH + L,), jnp.int32),
            pltpu.VMEM((B + CH + L,), jnp.int32),
            pltpu.VMEM((slab,), jnp.int32),
            pltpu.VMEM((CH,), jnp.int32),
            pltpu.VMEM((CH, DP), jnp.float32),
            pltpu.SemaphoreType.DMA,
        ],
    )
    kern(idx, h_new_pad, buf)


def _t_bwd_body(pad_ref, o_ref):
    d = o_ref.shape[0]
    o_ref[...] = jnp.transpose(pad_ref[...][:, :d])


def _tc_transpose_back(out_pad, D):
    M = out_pad.shape[0]
    BW = 16384
    grid = (pl.cdiv(M, BW),)
    return pl.pallas_call(
        _t_bwd_body,
        grid=grid,
        in_specs=[pl.BlockSpec((BW, 2 * D), lambda i: (i, 0))],
        out_specs=pl.BlockSpec((D, BW), lambda i: (0, i)),
        out_shape=jax.ShapeDtypeStruct((D, M), jnp.float32),
    )(out_pad)


def kernel(mem, idx, val, W_i, W_h, b_i, b_h):
    D = mem.shape[1]
    memT = jnp.transpose(mem)
    valT = jnp.transpose(val)
    mem_pad = _tc_transpose_pad(memT)
    h_pad = _sc_gather(idx, mem_pad)
    h_new_pad = _tc_gru(h_pad, valT, W_i, W_h, b_i, b_h)
    buf = jax.new_ref(mem_pad)
    _sc_scatter(buf, idx, h_new_pad)
    outT = _tc_transpose_back(buf[...], D)
    return jnp.transpose(outT)


# speedup vs baseline: 15.7742x; 1.0157x over previous
import functools

import jax
import jax.numpy as jnp
from jax import lax
from jax.experimental import pallas as pl
from jax.experimental.pallas import tpu as pltpu
from jax.experimental.pallas import tpu_sc as plsc

NC = 2
NS = 16
L = 16
NW = NC * NS
DP = 128
CH = 128


def _t_fwd_body(memT_ref, o_ref):
    t = jnp.transpose(memT_ref[...])
    o_ref[...] = jnp.concatenate([t, jnp.zeros_like(t)], axis=1)


def _tc_transpose_pad(memT):
    D, M = memT.shape
    BW = 32768
    grid = (pl.cdiv(M, BW),)
    return pl.pallas_call(
        _t_fwd_body,
        grid=grid,
        in_specs=[pl.BlockSpec((D, BW), lambda i: (0, i))],
        out_specs=pl.BlockSpec((BW, 2 * D), lambda i: (i, 0)),
        out_shape=jax.ShapeDtypeStruct((M, 2 * D), jnp.float32),
    )(memT)


def _gather_body(b_per_w, idx_hbm, pad_hbm, h_hbm, idx_v, rows_v, sem):
    cid = lax.axis_index("c")
    sid = lax.axis_index("s")
    wid = sid * NC + cid
    base = wid * b_per_w
    pltpu.sync_copy(idx_hbm.at[pl.ds(base, b_per_w)], idx_v)
    pltpu.async_copy(pad_hbm.at[idx_v], rows_v, sem).wait()
    pltpu.sync_copy(rows_v, h_hbm.at[pl.ds(base, b_per_w)])


def _sc_gather(idx, mem_pad):
    B = idx.shape[0]
    b_per_w = B // NW
    mesh = plsc.VectorSubcoreMesh(core_axis_name="c", subcore_axis_name="s")
    kern = pl.kernel(
        functools.partial(_gather_body, b_per_w),
        out_type=jax.ShapeDtypeStruct((B, DP), jnp.float32),
        mesh=mesh,
        compiler_params=pltpu.CompilerParams(needs_layout_passes=False),
        scratch_types=[
            pltpu.VMEM((b_per_w,), jnp.int32),
            pltpu.VMEM((b_per_w, DP), jnp.float32),
            pltpu.SemaphoreType.DMA,
        ],
    )
    return kern(idx, mem_pad)


def _gru_body(h_ref, valT_ref, wi_ref, wh_ref, bi_ref, bh_ref, o_ref):
    d = wi_ref.shape[0]
    h = h_ref[...][:, :d]
    v = jnp.transpose(valT_ref[...])
    gi = jnp.dot(v, wi_ref[...], preferred_element_type=jnp.float32) + bi_ref[...]
    gh = jnp.dot(h, wh_ref[...], preferred_element_type=jnp.float32) + bh_ref[...]
    r = jax.nn.sigmoid(gi[:, :d] + gh[:, :d])
    z = jax.nn.sigmoid(gi[:, d:2 * d] + gh[:, d:2 * d])
    n = jnp.tanh(gi[:, 2 * d:] + r * gh[:, 2 * d:])
    hn = (1.0 - z) * n + z * h
    o_ref[...] = jnp.concatenate([hn, jnp.zeros_like(hn)], axis=1)


def _tc_gru(h_pad, valT, W_i, W_h, b_i, b_h):
    B = h_pad.shape[0]
    D = W_i.shape[0]
    BLK = 1024
    grid = (B // BLK,)
    return pl.pallas_call(
        _gru_body,
        grid=grid,
        in_specs=[
            pl.BlockSpec((BLK, DP), lambda i: (i, 0)),
            pl.BlockSpec((D, BLK), lambda i: (0, i)),
            pl.BlockSpec((D, 3 * D), lambda i: (0, 0)),
            pl.BlockSpec((D, 3 * D), lambda i: (0, 0)),
            pl.BlockSpec((1, 3 * D), lambda i: (0, 0)),
            pl.BlockSpec((1, 3 * D), lambda i: (0, 0)),
        ],
        out_specs=pl.BlockSpec((BLK, DP), lambda i: (i, 0)),
        out_shape=jax.ShapeDtypeStruct((B, DP), jnp.float32),
    )(h_pad, valT, W_i, W_h, b_i.reshape(1, 3 * D), b_h.reshape(1, 3 * D))


def _scatter_body(M, B, slab, idx_hbm, hnew_hbm, buf_ref,
                  idxall_v, selpos_v, seltgt_v, winpos_v, alocal_v,
                  ctgt_v, crows_v, sem):
    cid = lax.axis_index("c")
    sid = lax.axis_index("s")
    wid = sid * NC + cid
    lo = wid * slab

    pltpu.sync_copy(idx_hbm, idxall_v.at[pl.ds(0, B)])

    def filt(k, off):
        v = idxall_v[pl.ds(k * L, L)]
        m = (v >= lo) & (v < lo + slab)
        pos = k * L + lax.iota(jnp.int32, L)
        pc = plsc.cumsum(m.astype(jnp.int32))
        plsc.store_scatter(selpos_v, [off + pc - 1], pos, mask=m)
        return off + jnp.max(pc)

    count = lax.fori_loop(0, B // L, filt, jnp.int32(0), unroll=4)

    @pl.when(count > 0)
    def _work():
        pv = plsc.load_gather(selpos_v, [jnp.full((L,), count - 1, jnp.int32)])
        for t in range(CH // L + 1):
            selpos_v[pl.ds(count + t * L, L)] = pv

        nv = (count + CH + L - 1) // L

        def canon1(i, _):
            p = selpos_v[pl.ds(i * L, L)]
            tv = plsc.load_gather(idxall_v, [p]) - lo
            seltgt_v[pl.ds(i * L, L)] = tv
            plsc.store_scatter(alocal_v, [tv], p)
            return 0

        lax.fori_loop(0, nv, canon1, 0)

        def canon2(i, _):
            tv = seltgt_v[pl.ds(i * L, L)]
            winpos_v[pl.ds(i * L, L)] = plsc.load_gather(alocal_v, [tv])
            return 0

        lax.fori_loop(0, nv, canon2, 0)

        def chunk(j, _):
            for k in range(CH // L):
                ctgt_v[pl.ds(k * L, L)] = (
                    seltgt_v[pl.ds(j * CH + k * L, L)] + lo)
            cw = winpos_v.at[pl.ds(j * CH, CH)]
            pltpu.async_copy(hnew_hbm.at[cw], crows_v, sem).wait()
            pltpu.sync_copy(crows_v, buf_ref.at[ctgt_v])
            return 0

        lax.fori_loop(0, (count + CH - 1) // CH, chunk, 0)


def _sc_scatter(buf, idx, h_new_pad):
    M = buf.shape[0]
    B = idx.shape[0]
    slab = M // NW
    mesh = plsc.VectorSubcoreMesh(core_axis_name="c", subcore_axis_name="s")
    kern = pl.kernel(
        functools.partial(_scatter_body, M, B, slab),
        out_type=(),
        mesh=mesh,
        compiler_params=pltpu.CompilerParams(needs_layout_passes=False),
        scratch_types=[
            pltpu.VMEM((B + L,), jnp.int32),
            pltpu.VMEM((B + CH + L,), jnp.int32),
            pltpu.VMEM((B + CH + L,), jnp.int32),
            pltpu.VMEM((B + CH + L,), jnp.int32),
            pltpu.VMEM((slab,), jnp.int32),
            pltpu.VMEM((CH,), jnp.int32),
            pltpu.VMEM((CH, DP), jnp.float32),
            pltpu.SemaphoreType.DMA,
        ],
    )
    kern(idx, h_new_pad, buf)


def _t_bwd_body(pad_ref, o_ref):
    d = o_ref.shape[0]
    o_ref[...] = jnp.transpose(pad_ref[...][:, :d])


def _tc_transpose_back(out_pad, D):
    M = out_pad.shape[0]
    BW = 32768
    grid = (pl.cdiv(M, BW),)
    return pl.pallas_call(
        _t_bwd_body,
        grid=grid,
        in_specs=[pl.BlockSpec((BW, 2 * D), lambda i: (i, 0))],
        out_specs=pl.BlockSpec((D, BW), lambda i: (0, i)),
        out_shape=jax.ShapeDtypeStruct((D, M), jnp.float32),
    )(out_pad)


def kernel(mem, idx, val, W_i, W_h, b_i, b_h):
    D = mem.shape[1]
    memT = jnp.transpose(mem)
    valT = jnp.transpose(val)
    mem_pad = _tc_transpose_pad(memT)
    h_pad = _sc_gather(idx, mem_pad)
    h_new_pad = _tc_gru(h_pad, valT, W_i, W_h, b_i, b_h)
    buf = jax.new_ref(mem_pad)
    _sc_scatter(buf, idx, h_new_pad)
    outT = _tc_transpose_back(buf[...], D)
    return jnp.transpose(outT)
